# baseline (speedup 1.0000x reference)
import functools

import jax
import jax.numpy as jnp
from jax import lax
from jax.experimental import pallas as pl
from jax.experimental.pallas import tpu as pltpu
from jax.experimental.pallas import tpu_sc as plsc

BLK = 64
D = 1024
W = 512
L = 16
NW = 32
NS = 192
UPW = NS // NW
TC_G = 8


def _sc_body(states, maskr, diff, mean, cmask,
             xb0, xb1, mall, meanbuf, cmbuf, sem0, sem1):
    wid = lax.axis_index("s") * 2 + lax.axis_index("c")
    lanes = lax.iota(jnp.int32, L)
    base = wid * UPW

    pltpu.sync_copy(maskr.at[pl.ds(base * BLK, UPW * BLK)], mall)
    pltpu.async_copy(states.at[base, :, pl.ds(0, W)], xb0, sem0)

    def compute(buf, ms, rcp, mo):
        def col(c, carry):
            o = pl.multiple_of(c * L, L)
            acc = [jnp.zeros((L,), jnp.float32) for _ in range(4)]
            for r in range(BLK):
                xm = buf[r, pl.ds(o, L)] * ms[r]
                buf[r, pl.ds(o, L)] = xm
                acc[r % 4] = acc[r % 4] + xm
            mc = ((acc[0] + acc[1]) + (acc[2] + acc[3])) * rcp
            meanbuf[pl.ds(mo + o, L)] = mc
            for r in range(BLK):
                buf[r, pl.ds(o, L)] = mc - buf[r, pl.ds(o, L)]
            return carry

        lax.fori_loop(0, W // L, col, 0)

    def unit(i, cmvec):
        u = base + i
        mchunks = [mall[pl.ds(i * BLK + k * L, L)] for k in range(BLK // L)]
        ms = [mchunks[r // L][r % L] for r in range(BLK)]
        cnt = functools.reduce(lambda a, b: a + b, ms)
        denom = jnp.full((L,), cnt, jnp.float32) + 1e-4
        rcp = 1.0 / denom

        pltpu.make_async_copy(states.at[u, :, pl.ds(0, W)], xb0, sem0).wait()
        pltpu.async_copy(states.at[u, :, pl.ds(W, W)], xb1, sem1)
        compute(xb0, ms, rcp, 0)
        pltpu.sync_copy(xb0, diff.at[u, :, pl.ds(0, W)])

        pltpu.make_async_copy(states.at[u, :, pl.ds(W, W)], xb1, sem1).wait()

        @pl.when(i + 1 < UPW)
        def _():
            pltpu.async_copy(states.at[u + 1, :, pl.ds(0, W)], xb0, sem0)

        compute(xb1, ms, rcp, W)
        pltpu.sync_copy(xb1, diff.at[u, :, pl.ds(W, W)])
        pltpu.sync_copy(meanbuf, mean.at[u])

        cmval = (cnt > 0.0).astype(jnp.float32)
        return jnp.where(lanes == i, cmval, cmvec)

    cmvec = lax.fori_loop(0, UPW, unit, jnp.zeros((L,), jnp.float32))
    cmbuf[...] = cmvec
    pltpu.sync_copy(cmbuf.at[pl.ds(0, 8)], cmask.at[pl.ds(wid * 8, 8)])


def _run_sc(sr, mr):
    n = sr.shape[0]
    mesh = plsc.VectorSubcoreMesh(core_axis_name="c", subcore_axis_name="s")
    return pl.kernel(
        _sc_body,
        mesh=mesh,
        out_type=[
            jax.ShapeDtypeStruct((NS, BLK, D), jnp.float32),
            jax.ShapeDtypeStruct((NS, D), jnp.float32),
            jax.ShapeDtypeStruct((NW * 8,), jnp.float32),
        ],
        scratch_types=[
            pltpu.VMEM((BLK, W), jnp.float32),
            pltpu.VMEM((BLK, W), jnp.float32),
            pltpu.VMEM((UPW * BLK,), jnp.float32),
            pltpu.VMEM((D,), jnp.float32),
            pltpu.VMEM((L,), jnp.float32),
            pltpu.SemaphoreType.DMA,
            pltpu.SemaphoreType.DMA,
        ],
    )(sr, mr)


def _tc_body(x_ref, m_ref, diff_ref, mean_ref, cmask_ref):
    g = TC_G
    x = x_ref[...]
    mc = m_ref[...]
    xm = x * mc
    xr = xm.reshape(g, BLK, D)
    s = jnp.sum(xr, axis=1)
    cnt = jnp.sum(mc.reshape(g, BLK, 1), axis=1)
    mean = s / (cnt + 1e-4)
    diff_ref[...] = (mean[:, None, :] - xr).reshape(g * BLK, D)
    mean_ref[...] = mean
    cmask_ref[...] = (cnt > 0).astype(x.dtype)


def _run_tc(x2d, mcol):
    n_tc = x2d.shape[0] // BLK - NS
    off = NS // TC_G
    grid = (n_tc // TC_G,)
    return pl.pallas_call(
        _tc_body,
        grid=grid,
        in_specs=[
            pl.BlockSpec((TC_G * BLK, D), lambda j: (off + j, 0)),
            pl.BlockSpec((TC_G * BLK, 1), lambda j: (off + j, 0)),
        ],
        out_specs=[
            pl.BlockSpec((TC_G * BLK, D), lambda j: (j, 0)),
            pl.BlockSpec((TC_G, D), lambda j: (j, 0)),
            pl.BlockSpec((TC_G, 1), lambda j: (j, 0)),
        ],
        out_shape=[
            jax.ShapeDtypeStruct((n_tc * BLK, D), jnp.float32),
            jax.ShapeDtypeStruct((n_tc, D), jnp.float32),
            jax.ShapeDtypeStruct((n_tc, 1), jnp.float32),
        ],
        compiler_params=pltpu.CompilerParams(
            dimension_semantics=("parallel",),
        ),
    )(x2d, mcol)


@jax.jit
def _run(states, mask):
    b, f, d = states.shape
    nb = f // BLK
    n = b * nb
    sr = states.reshape(n, BLK, d)
    mr = mask.reshape(n, BLK)
    x2d = states.reshape(n * BLK, d)
    mcol = mask.reshape(n * BLK, 1)

    diff_sc, mean_sc, cm_sc = _run_sc(sr, mask.reshape(n * BLK))
    diff_tc, mean_tc, cm_tc = _run_tc(x2d, mcol)

    diff = jnp.concatenate([diff_sc.reshape(NS * BLK, d), diff_tc], axis=0)
    mean = jnp.concatenate([mean_sc, mean_tc], axis=0)
    cmask = jnp.concatenate(
        [cm_sc.reshape(NW, 8)[:, :UPW].reshape(NS), cm_tc.reshape(n - NS)], axis=0)
    return diff, mean, cmask


def kernel(fine_token_states, fine_token_mask):
    b, f, d = fine_token_states.shape
    nb = f // BLK
    diff, mean, cmask = _run(fine_token_states, fine_token_mask)
    indice = jnp.broadcast_to(jnp.arange(nb, dtype=jnp.int32)[None, :], (b, nb))
    return (mean.reshape(b, nb, d), cmask.reshape(b, nb),
            diff.reshape(b, nb, BLK, d), indice)
